# Initial kernel scaffold; baseline (speedup 1.0000x reference)
import jax
import jax.numpy as jnp
from jax.experimental import pallas as pl


def kernel(x, trg_edge, att_l, att_r, W_pred, b_pred):
    raise NotImplementedError("write your pallas kernel here")


# speedup vs baseline: 14.9577x; 14.9577x over previous
import functools

import jax
import jax.numpy as jnp
from jax import lax
from jax.experimental import pallas as pl
from jax.experimental.pallas import tpu as pltpu
from jax.experimental.pallas import tpu_sc as plsc

N = 10000
E = 320000
D = 128
DQ = D // 4
EPS = 0.5

NC = 2
NS = 16
EPW = E // NS
K = 80
NCHUNK = EPW // K
NP = 10240
RPT = NP // NS
RPN = N // NS

_MESH = plsc.VectorSubcoreMesh(core_axis_name="c", subcore_axis_name="s")
_SC_PARAMS = pltpu.CompilerParams(needs_layout_passes=False,
                                  use_tc_tiling_on_sc=False)


def _stable_tanh(z):
    e = jnp.exp(-2.0 * jnp.abs(z))
    return jnp.sign(z) * (1.0 - e) / (1.0 + e)


@functools.partial(
    pl.kernel,
    out_type=jax.ShapeDtypeStruct((NC, NS, NP), jnp.float32),
    mesh=_MESH,
    compiler_params=_SC_PARAMS,
    scratch_types=[
        pltpu.VMEM((NCHUNK, K), jnp.int32),
        pltpu.VMEM((NP,), jnp.float32),
    ],
)
def _sc_deg(edge_hbm, out_hbm, col2d, tbl):
    cid = lax.axis_index("c")
    sid = lax.axis_index("s")

    pltpu.sync_copy(edge_hbm.at[1, sid], col2d)

    zero16 = jnp.zeros((16,), jnp.float32)

    @pl.loop(0, NP // 16)
    def _(i):
        tbl[pl.ds(i * 16, 16)] = zero16

    jbase = cid * (NCHUNK // 2)

    @pl.loop(0, NCHUNK // 2)
    def _(j):
        for v in range(K // 16):
            ci = col2d[jbase + j, pl.ds(v * 16, 16)]
            cnt, last = plsc.scan_count(ci)
            plsc.addupdate_scatter(tbl, [ci], cnt.astype(jnp.float32),
                                   mask=last)

    pltpu.sync_copy(tbl, out_hbm.at[cid, sid])


@functools.partial(
    pl.kernel,
    out_type=jax.ShapeDtypeStruct((4, N, DQ), jnp.float32),
    mesh=_MESH,
    compiler_params=_SC_PARAMS,
    scratch_types=[
        pltpu.VMEM((NCHUNK, K), jnp.int32),
        pltpu.VMEM((NCHUNK, K), jnp.int32),
        pltpu.VMEM((NP,), jnp.float32),
        pltpu.VMEM((NP,), jnp.float32),
        pltpu.VMEM((NP,), jnp.float32),
        pltpu.VMEM((NCHUNK, K), jnp.float32),
        pltpu.VMEM((K, DQ), jnp.float32),
        pltpu.VMEM((125, DQ), jnp.float32),
        pltpu.VMEM_SHARED((N, DQ), jnp.float32),
        pltpu.SemaphoreType.DMA,
    ],
)
def _sc_edge(xq_hbm, edge_hbm, al_hbm, ar_hbm, dis_hbm, out_hbm,
             row2d, col2d, al_v, ar_v, dis_v, c2d, rows_v, zbuf, acc, sem):
    cid = lax.axis_index("c")
    sid = lax.axis_index("s")

    pltpu.sync_copy(edge_hbm.at[0, sid], row2d)
    pltpu.sync_copy(edge_hbm.at[1, sid], col2d)
    pltpu.sync_copy(al_hbm, al_v.at[pl.ds(0, N)])
    pltpu.sync_copy(ar_hbm, ar_v.at[pl.ds(0, N)])
    pltpu.sync_copy(dis_hbm, dis_v.at[pl.ds(0, N)])

    zero16 = jnp.zeros((16,), jnp.float32)

    @pl.loop(0, 125)
    def _(i):
        for d in range(DQ // 16):
            zbuf[i, pl.ds(d * 16, 16)] = zero16

    @pl.loop(0, NCHUNK)
    def _(j):
        for v in range(K // 16):
            sl = pl.ds(v * 16, 16)
            ri = row2d[j, sl]
            ci = col2d[j, sl]
            av = plsc.load_gather(al_v, [ri])
            bv = plsc.load_gather(ar_v, [ci])
            dr = plsc.load_gather(dis_v, [ri])
            dc = plsc.load_gather(dis_v, [ci])
            c2d[j, sl] = _stable_tanh(av + bv) * dr * dc

    for p in range(2):
        qid = 2 * cid + p

        @pl.loop(0, RPN // 125)
        def _(i):
            pltpu.sync_copy(zbuf, acc.at[pl.ds(sid * RPN + i * 125, 125)])

        plsc.subcore_barrier()

        @pl.loop(0, NCHUNK)
        def _(j):
            pltpu.async_copy(xq_hbm.at[qid].at[row2d.at[j]], rows_v,
                             sem).wait()

            @pl.loop(0, K // 16)
            def _(v):
                cv = c2d[j, pl.ds(v * 16, 16)]
                base = v * 16
                for l in range(16):
                    cj = cv[l]
                    for d in range(DQ // 16):
                        sl = pl.ds(d * 16, 16)
                        rows_v[base + l, sl] = rows_v[base + l, sl] * cj

            pltpu.sync_copy(rows_v, acc.at[col2d.at[j]], add=True)

        plsc.subcore_barrier()
        pltpu.sync_copy(acc.at[pl.ds(sid * RPN, RPN)],
                        out_hbm.at[qid, pl.ds(sid * RPN, RPN)])


BLK = 2000
GRID = N // BLK


def _tcdeg_body(degp_ref, deg_ref):
    deg_ref[...] = (jnp.sum(degp_ref[...], axis=(0, 1)) + 1.0)[:, None]


def _tcdeg(degp):
    return pl.pallas_call(
        _tcdeg_body,
        grid=(1,),
        in_specs=[pl.BlockSpec((NC, NS, NP), lambda i: (0, 0, 0))],
        out_specs=pl.BlockSpec((NP, 1), lambda i: (0, 0)),
        out_shape=jax.ShapeDtypeStruct((NP, 1), jnp.float32),
    )(degp)


def _tc1_body(x_ref, deg_ref, att2_ref,
              al_ref, ar_ref, dis_ref, s1_ref, xq_ref):
    x = x_ref[...]
    aa = jnp.dot(x, att2_ref[...], preferred_element_type=jnp.float32)
    al = aa[:, 0]
    ar = aa[:, 1]
    deg = deg_ref[:, 0]
    dis = lax.rsqrt(deg)
    s1 = jnp.tanh(al + ar) / deg
    al_ref[...] = al[:, None]
    ar_ref[...] = ar[:, None]
    dis_ref[...] = dis[:, None]
    s1_ref[...] = s1[:, None]
    for q in range(4):
        xq_ref[q] = x[:, q * DQ:(q + 1) * DQ]


def _tc1(x, deg, att2):
    return pl.pallas_call(
        _tc1_body,
        grid=(GRID,),
        in_specs=[
            pl.BlockSpec((BLK, D), lambda i: (i, 0)),
            pl.BlockSpec((BLK, 1), lambda i: (i, 0)),
            pl.BlockSpec((D, 2), lambda i: (0, 0)),
        ],
        out_specs=[
            pl.BlockSpec((BLK, 1), lambda i: (i, 0)),
            pl.BlockSpec((BLK, 1), lambda i: (i, 0)),
            pl.BlockSpec((BLK, 1), lambda i: (i, 0)),
            pl.BlockSpec((BLK, 1), lambda i: (i, 0)),
            pl.BlockSpec((4, BLK, DQ), lambda i: (0, i, 0)),
        ],
        out_shape=[jax.ShapeDtypeStruct((N, 1), jnp.float32)] * 4
        + [jax.ShapeDtypeStruct((4, N, DQ), jnp.float32)],
    )(x, deg, att2)


def _tcd1_body(p_ref, x_ref, s1_ref, dis_ref, att2_ref,
               trgq_ref, al2_ref, ar2_ref, s2_ref):
    c0 = (s1_ref[...] + EPS) * x_ref[...]
    trg = jnp.concatenate([p_ref[q] for q in range(4)], axis=1) + c0
    aa = jnp.dot(trg, att2_ref[...], preferred_element_type=jnp.float32)
    al2 = aa[:, 0]
    ar2 = aa[:, 1]
    dis = dis_ref[:, 0]
    s2 = jnp.tanh(al2 + ar2) * dis * dis
    al2_ref[...] = al2[:, None]
    ar2_ref[...] = ar2[:, None]
    s2_ref[...] = s2[:, None]
    for q in range(4):
        trgq_ref[q] = trg[:, q * DQ:(q + 1) * DQ]


def _tcd1(p, x, s1, dis, att2):
    return pl.pallas_call(
        _tcd1_body,
        grid=(GRID,),
        in_specs=[
            pl.BlockSpec((4, BLK, DQ), lambda i: (0, i, 0)),
            pl.BlockSpec((BLK, D), lambda i: (i, 0)),
            pl.BlockSpec((BLK, 1), lambda i: (i, 0)),
            pl.BlockSpec((BLK, 1), lambda i: (i, 0)),
            pl.BlockSpec((D, 2), lambda i: (0, 0)),
        ],
        out_specs=[
            pl.BlockSpec((4, BLK, DQ), lambda i: (0, i, 0)),
            pl.BlockSpec((BLK, 1), lambda i: (i, 0)),
            pl.BlockSpec((BLK, 1), lambda i: (i, 0)),
            pl.BlockSpec((BLK, 1), lambda i: (i, 0)),
        ],
        out_shape=[
            jax.ShapeDtypeStruct((4, N, DQ), jnp.float32),
            jax.ShapeDtypeStruct((N, 1), jnp.float32),
            jax.ShapeDtypeStruct((N, 1), jnp.float32),
            jax.ShapeDtypeStruct((N, 1), jnp.float32),
        ],
    )(p, x, s1, dis, att2)


def _tcd2_body(q_ref, trgq_ref, x_ref, s2_ref, wt_ref, b_ref, out_ref):
    trg = jnp.concatenate([trgq_ref[i] for i in range(4)], axis=1)
    q = jnp.concatenate([q_ref[i] for i in range(4)], axis=1)
    h2 = q + s2_ref[...] * trg + EPS * x_ref[...]
    out_ref[...] = (jnp.dot(h2, wt_ref[...], preferred_element_type=jnp.float32)
                    + b_ref[...])


def _tcd2(q, trgq, x, s2, wt, b):
    return pl.pallas_call(
        _tcd2_body,
        grid=(GRID,),
        in_specs=[
            pl.BlockSpec((4, BLK, DQ), lambda i: (0, i, 0)),
            pl.BlockSpec((4, BLK, DQ), lambda i: (0, i, 0)),
            pl.BlockSpec((BLK, D), lambda i: (i, 0)),
            pl.BlockSpec((BLK, 1), lambda i: (i, 0)),
            pl.BlockSpec((D, 2), lambda i: (0, 0)),
            pl.BlockSpec((1, 2), lambda i: (0, 0)),
        ],
        out_specs=pl.BlockSpec((BLK, 2), lambda i: (i, 0)),
        out_shape=jax.ShapeDtypeStruct((N, 2), jnp.float32),
    )(q, trgq, x, s2, wt, b)


def kernel(x, trg_edge, att_l, att_r, W_pred, b_pred):
    edge4d = trg_edge.reshape(2, NS, NCHUNK, K)
    att2 = jnp.stack([att_l, att_r], axis=1)
    degp = _sc_deg(edge4d)
    deg = _tcdeg(degp)
    al, ar, dis, s1, xq = _tc1(x, deg, att2)
    dis1 = dis.reshape(N)
    p = _sc_edge(xq, edge4d, al.reshape(N), ar.reshape(N), dis1)
    trgq, al2, ar2, s2 = _tcd1(p, x, s1, dis, att2)
    q = _sc_edge(trgq, edge4d, al2.reshape(N), ar2.reshape(N), dis1)
    return _tcd2(q, trgq, x, s2, W_pred.T, b_pred.reshape(1, 2))


# speedup vs baseline: 26.2175x; 1.7528x over previous
import functools

import jax
import jax.numpy as jnp
from jax import lax
from jax.experimental import pallas as pl
from jax.experimental.pallas import tpu as pltpu
from jax.experimental.pallas import tpu_sc as plsc

N = 10000
E = 320000
D = 128
DQ = D // 4
EPS = 0.5

NC = 2
NS = 16
EPW = E // NS
K = 80
NCHUNK = EPW // K
NP = 10240
RPT = NP // NS
RPN = N // NS

_MESH = plsc.VectorSubcoreMesh(core_axis_name="c", subcore_axis_name="s")
_SC_PARAMS = pltpu.CompilerParams(needs_layout_passes=False,
                                  use_tc_tiling_on_sc=False)


def _stable_tanh(z):
    e = jnp.exp(-2.0 * jnp.abs(z))
    return jnp.sign(z) * (1.0 - e) / (1.0 + e)


@functools.partial(
    pl.kernel,
    out_type=jax.ShapeDtypeStruct((NC, NS, NP), jnp.float32),
    mesh=_MESH,
    compiler_params=_SC_PARAMS,
    scratch_types=[
        pltpu.VMEM((NCHUNK, K), jnp.int32),
        pltpu.VMEM((NP,), jnp.float32),
    ],
)
def _sc_deg(edge_hbm, out_hbm, col2d, tbl):
    cid = lax.axis_index("c")
    sid = lax.axis_index("s")

    pltpu.sync_copy(edge_hbm.at[1, sid], col2d)

    zero16 = jnp.zeros((16,), jnp.float32)

    @pl.loop(0, NP // 16)
    def _(i):
        tbl[pl.ds(i * 16, 16)] = zero16

    jbase = cid * (NCHUNK // 2)

    @pl.loop(0, NCHUNK // 2)
    def _(j):
        for v in range(K // 16):
            ci = col2d[jbase + j, pl.ds(v * 16, 16)]
            cnt, last = plsc.scan_count(ci)
            plsc.addupdate_scatter(tbl, [ci], cnt.astype(jnp.float32),
                                   mask=last)

    pltpu.sync_copy(tbl, out_hbm.at[cid, sid])


@functools.partial(
    pl.kernel,
    out_type=jax.ShapeDtypeStruct((4, N, DQ), jnp.float32),
    mesh=_MESH,
    compiler_params=_SC_PARAMS,
    scratch_types=[
        pltpu.VMEM((NCHUNK, K), jnp.int32),
        pltpu.VMEM((NCHUNK, K), jnp.int32),
        pltpu.VMEM((NP,), jnp.float32),
        pltpu.VMEM((NP,), jnp.float32),
        pltpu.VMEM((NCHUNK, K), jnp.float32),
        pltpu.VMEM((K, DQ), jnp.float32),
        pltpu.VMEM((K, DQ), jnp.float32),
        pltpu.VMEM((125, DQ), jnp.float32),
        pltpu.VMEM_SHARED((N, DQ), jnp.float32),
        pltpu.SemaphoreType.DMA,
        pltpu.SemaphoreType.DMA,
        pltpu.SemaphoreType.DMA,
        pltpu.SemaphoreType.DMA,
    ],
)
def _sc_edge(xq_hbm, edge_hbm, al_hbm, ar_hbm, out_hbm,
             row2d, col2d, al_v, ar_v, c2d, rows0, rows1, zbuf, acc,
             gsem0, gsem1, ssem0, ssem1):
    cid = lax.axis_index("c")
    sid = lax.axis_index("s")

    rows = (rows0, rows1)
    gsem = (gsem0, gsem1)
    ssem = (ssem0, ssem1)

    pltpu.sync_copy(edge_hbm.at[0, sid], row2d)
    pltpu.sync_copy(edge_hbm.at[1, sid], col2d)
    pltpu.sync_copy(al_hbm, al_v.at[pl.ds(0, N)])
    pltpu.sync_copy(ar_hbm, ar_v.at[pl.ds(0, N)])

    zero16 = jnp.zeros((16,), jnp.float32)

    @pl.loop(0, 125)
    def _(i):
        for d in range(DQ // 16):
            zbuf[i, pl.ds(d * 16, 16)] = zero16

    def coeffs(j):
        for v in range(K // 16):
            sl = pl.ds(v * 16, 16)
            ri = row2d[j, sl]
            ci = col2d[j, sl]
            av = plsc.load_gather(al_v, [ri])
            bv = plsc.load_gather(ar_v, [ci])
            c2d[j, sl] = _stable_tanh(av + bv)

    def gather(j, b, q):
        return pltpu.make_async_copy(
            xq_hbm.at[q].at[row2d.at[j]], rows[b], gsem[b])

    def scatter(j, b):
        return pltpu.make_async_copy(rows[b], acc.at[col2d.at[j]], ssem[b])

    for p in range(2):
        qid = 2 * cid + p

        @pl.loop(0, RPN // 125)
        def _(i):
            pltpu.sync_copy(zbuf, acc.at[pl.ds(sid * RPN + i * 125, 125)])

        plsc.subcore_barrier()

        gather(0, 0, qid).start()

        @pl.loop(0, NCHUNK, step=2)
        def _(j):
            for b in range(2):
                jj = j + b

                @pl.when(jj >= 1)
                def _():
                    scatter(jj - 1, 1 - b).wait()

                @pl.when(jj + 1 < NCHUNK)
                def _():
                    gather(jj + 1, 1 - b, qid).start()

                if p == 0:
                    coeffs(jj)

                gather(jj, b, qid).wait()
                rv = rows[b]

                @pl.loop(0, K // 16)
                def _(v):
                    cv = c2d[jj, pl.ds(v * 16, 16)]
                    base = v * 16
                    for l in range(16):
                        cj = cv[l]
                        for d in range(DQ // 16):
                            sl = pl.ds(d * 16, 16)
                            rv[base + l, sl] = rv[base + l, sl] * cj

                scatter(jj, b).start(add=True)

        scatter(NCHUNK - 1, 1).wait()

        plsc.subcore_barrier()
        pltpu.sync_copy(acc.at[pl.ds(sid * RPN, RPN)],
                        out_hbm.at[qid, pl.ds(sid * RPN, RPN)])


BLK = 2000
GRID = N // BLK


def _tcdeg_body(degp_ref, deg_ref):
    deg_ref[...] = (jnp.sum(degp_ref[...], axis=(0, 1)) + 1.0)[:, None]


def _tcdeg(degp):
    return pl.pallas_call(
        _tcdeg_body,
        grid=(1,),
        in_specs=[pl.BlockSpec((NC, NS, NP), lambda i: (0, 0, 0))],
        out_specs=pl.BlockSpec((NP, 1), lambda i: (0, 0)),
        out_shape=jax.ShapeDtypeStruct((NP, 1), jnp.float32),
    )(degp)


def _tc1_body(x_ref, deg_ref, att2_ref,
              al_ref, ar_ref, dis_ref, s1_ref, xq_ref):
    x = x_ref[...]
    aa = jnp.dot(x, att2_ref[...], preferred_element_type=jnp.float32)
    al = aa[:, 0]
    ar = aa[:, 1]
    deg = deg_ref[:, 0]
    dis = lax.rsqrt(deg)
    s1 = jnp.tanh(al + ar) / deg
    al_ref[...] = al[:, None]
    ar_ref[...] = ar[:, None]
    dis_ref[...] = dis[:, None]
    s1_ref[...] = s1[:, None]
    xs = x * dis[:, None]
    for q in range(4):
        xq_ref[q] = xs[:, q * DQ:(q + 1) * DQ]


def _tc1(x, deg, att2):
    return pl.pallas_call(
        _tc1_body,
        grid=(GRID,),
        in_specs=[
            pl.BlockSpec((BLK, D), lambda i: (i, 0)),
            pl.BlockSpec((BLK, 1), lambda i: (i, 0)),
            pl.BlockSpec((D, 2), lambda i: (0, 0)),
        ],
        out_specs=[
            pl.BlockSpec((BLK, 1), lambda i: (i, 0)),
            pl.BlockSpec((BLK, 1), lambda i: (i, 0)),
            pl.BlockSpec((BLK, 1), lambda i: (i, 0)),
            pl.BlockSpec((BLK, 1), lambda i: (i, 0)),
            pl.BlockSpec((4, BLK, DQ), lambda i: (0, i, 0)),
        ],
        out_shape=[jax.ShapeDtypeStruct((N, 1), jnp.float32)] * 4
        + [jax.ShapeDtypeStruct((4, N, DQ), jnp.float32)],
    )(x, deg, att2)


def _tcd1_body(p_ref, x_ref, s1_ref, dis_ref, att2_ref,
               trgq_ref, al2_ref, ar2_ref, s2_ref):
    dis = dis_ref[...]
    c0 = (s1_ref[...] + EPS) * x_ref[...]
    trg = dis * jnp.concatenate([p_ref[q] for q in range(4)], axis=1) + c0
    aa = jnp.dot(trg, att2_ref[...], preferred_element_type=jnp.float32)
    al2 = aa[:, 0]
    ar2 = aa[:, 1]
    d1 = dis[:, 0]
    s2 = jnp.tanh(al2 + ar2) * d1
    al2_ref[...] = al2[:, None]
    ar2_ref[...] = ar2[:, None]
    s2_ref[...] = s2[:, None]
    trgs = trg * dis
    for q in range(4):
        trgq_ref[q] = trgs[:, q * DQ:(q + 1) * DQ]


def _tcd1(p, x, s1, dis, att2):
    return pl.pallas_call(
        _tcd1_body,
        grid=(GRID,),
        in_specs=[
            pl.BlockSpec((4, BLK, DQ), lambda i: (0, i, 0)),
            pl.BlockSpec((BLK, D), lambda i: (i, 0)),
            pl.BlockSpec((BLK, 1), lambda i: (i, 0)),
            pl.BlockSpec((BLK, 1), lambda i: (i, 0)),
            pl.BlockSpec((D, 2), lambda i: (0, 0)),
        ],
        out_specs=[
            pl.BlockSpec((4, BLK, DQ), lambda i: (0, i, 0)),
            pl.BlockSpec((BLK, 1), lambda i: (i, 0)),
            pl.BlockSpec((BLK, 1), lambda i: (i, 0)),
            pl.BlockSpec((BLK, 1), lambda i: (i, 0)),
        ],
        out_shape=[
            jax.ShapeDtypeStruct((4, N, DQ), jnp.float32),
            jax.ShapeDtypeStruct((N, 1), jnp.float32),
            jax.ShapeDtypeStruct((N, 1), jnp.float32),
            jax.ShapeDtypeStruct((N, 1), jnp.float32),
        ],
    )(p, x, s1, dis, att2)


def _tcd2_body(q_ref, trgq_ref, x_ref, s2_ref, dis_ref, wt_ref, b_ref,
               out_ref):
    trgs = jnp.concatenate([trgq_ref[i] for i in range(4)], axis=1)
    q = jnp.concatenate([q_ref[i] for i in range(4)], axis=1)
    h2 = dis_ref[...] * q + s2_ref[...] * trgs + EPS * x_ref[...]
    out_ref[...] = (jnp.dot(h2, wt_ref[...], preferred_element_type=jnp.float32)
                    + b_ref[...])


def _tcd2(q, trgq, x, s2, dis, wt, b):
    return pl.pallas_call(
        _tcd2_body,
        grid=(GRID,),
        in_specs=[
            pl.BlockSpec((4, BLK, DQ), lambda i: (0, i, 0)),
            pl.BlockSpec((4, BLK, DQ), lambda i: (0, i, 0)),
            pl.BlockSpec((BLK, D), lambda i: (i, 0)),
            pl.BlockSpec((BLK, 1), lambda i: (i, 0)),
            pl.BlockSpec((BLK, 1), lambda i: (i, 0)),
            pl.BlockSpec((D, 2), lambda i: (0, 0)),
            pl.BlockSpec((1, 2), lambda i: (0, 0)),
        ],
        out_specs=pl.BlockSpec((BLK, 2), lambda i: (i, 0)),
        out_shape=jax.ShapeDtypeStruct((N, 2), jnp.float32),
    )(q, trgq, x, s2, dis, wt, b)


def kernel(x, trg_edge, att_l, att_r, W_pred, b_pred):
    edge4d = trg_edge.reshape(2, NS, NCHUNK, K)
    att2 = jnp.stack([att_l, att_r], axis=1)
    degp = _sc_deg(edge4d)
    deg = _tcdeg(degp)
    al, ar, dis, s1, xq = _tc1(x, deg, att2)
    p = _sc_edge(xq, edge4d, al.reshape(N), ar.reshape(N))
    trgq, al2, ar2, s2 = _tcd1(p, x, s1, dis, att2)
    q = _sc_edge(trgq, edge4d, al2.reshape(N), ar2.reshape(N))
    return _tcd2(q, trgq, x, s2, dis, W_pred.T, b_pred.reshape(1, 2))


# speedup vs baseline: 32.9206x; 1.2557x over previous
import functools

import jax
import jax.numpy as jnp
from jax import lax
from jax.experimental import pallas as pl
from jax.experimental.pallas import tpu as pltpu
from jax.experimental.pallas import tpu_sc as plsc

N = 10000
E = 320000
D = 128
DQ = D // 4
EPS = 0.5

NC = 2
NS = 16
EPW = E // NS
K = 80
NCHUNK = EPW // K
NP = 10240
RPT = NP // NS
RPN = N // NS

_MESH = plsc.VectorSubcoreMesh(core_axis_name="c", subcore_axis_name="s")
_SC_PARAMS = pltpu.CompilerParams(needs_layout_passes=False,
                                  use_tc_tiling_on_sc=False)


def _stable_tanh(z):
    e = jnp.exp(-2.0 * jnp.abs(z))
    return jnp.sign(z) * (1.0 - e) / (1.0 + e)


@functools.partial(
    pl.kernel,
    out_type=jax.ShapeDtypeStruct((NC, NS, NP), jnp.float32),
    mesh=_MESH,
    compiler_params=_SC_PARAMS,
    scratch_types=[
        pltpu.VMEM((NCHUNK, K), jnp.int32),
        pltpu.VMEM((NP,), jnp.float32),
    ],
)
def _sc_deg(edge_hbm, out_hbm, col2d, tbl):
    cid = lax.axis_index("c")
    sid = lax.axis_index("s")

    pltpu.sync_copy(edge_hbm.at[1, sid], col2d)

    zero16 = jnp.zeros((16,), jnp.float32)

    @pl.loop(0, NP // 16)
    def _(i):
        tbl[pl.ds(i * 16, 16)] = zero16

    jbase = cid * (NCHUNK // 2)

    @pl.loop(0, NCHUNK // 2)
    def _(j):
        for v in range(K // 16):
            ci = col2d[jbase + j, pl.ds(v * 16, 16)]
            cnt, last = plsc.scan_count(ci)
            plsc.addupdate_scatter(tbl, [ci], cnt.astype(jnp.float32),
                                   mask=last)

    pltpu.sync_copy(tbl, out_hbm.at[cid, sid])


@functools.partial(
    pl.kernel,
    out_type=jax.ShapeDtypeStruct((4, N, DQ), jnp.float32),
    mesh=_MESH,
    compiler_params=_SC_PARAMS,
    scratch_types=[
        pltpu.VMEM((NCHUNK, K), jnp.int32),
        pltpu.VMEM((NCHUNK, K), jnp.int32),
        pltpu.VMEM((NP,), jnp.float32),
        pltpu.VMEM((NP,), jnp.float32),
        pltpu.VMEM((NCHUNK, K), jnp.float32),
        pltpu.VMEM((K, DQ), jnp.float32),
        pltpu.VMEM((K, DQ), jnp.float32),
        pltpu.VMEM((125, DQ), jnp.float32),
        pltpu.VMEM_SHARED((N, DQ), jnp.float32),
        pltpu.VMEM_SHARED((N, DQ), jnp.float32),
        pltpu.SemaphoreType.DMA,
        pltpu.SemaphoreType.DMA,
        pltpu.SemaphoreType.DMA,
        pltpu.SemaphoreType.DMA,
    ],
)
def _sc_edge(xq_hbm, edge_hbm, al_hbm, ar_hbm, out_hbm,
             row2d, col2d, al_v, ar_v, c2d, rows0, rows1, zbuf, acc, src,
             gsem0, gsem1, ssem0, ssem1):
    cid = lax.axis_index("c")
    sid = lax.axis_index("s")

    rows = (rows0, rows1)
    gsem = (gsem0, gsem1)
    ssem = (ssem0, ssem1)

    pltpu.sync_copy(edge_hbm.at[0, sid], row2d)
    pltpu.sync_copy(edge_hbm.at[1, sid], col2d)
    pltpu.sync_copy(al_hbm, al_v.at[pl.ds(0, N)])
    pltpu.sync_copy(ar_hbm, ar_v.at[pl.ds(0, N)])

    zero16 = jnp.zeros((16,), jnp.float32)

    @pl.loop(0, 125)
    def _(i):
        for d in range(DQ // 16):
            zbuf[i, pl.ds(d * 16, 16)] = zero16

    def coeffs(j):
        for v in range(K // 16):
            sl = pl.ds(v * 16, 16)
            ri = row2d[j, sl]
            ci = col2d[j, sl]
            av = plsc.load_gather(al_v, [ri])
            bv = plsc.load_gather(ar_v, [ci])
            c2d[j, sl] = _stable_tanh(av + bv)

    def gather(j, b):
        return pltpu.make_async_copy(src.at[row2d.at[j]], rows[b], gsem[b])

    def scatter(j, b):
        return pltpu.make_async_copy(rows[b], acc.at[col2d.at[j]], ssem[b])

    for p in range(2):
        qid = 2 * cid + p

        pltpu.sync_copy(xq_hbm.at[qid, pl.ds(sid * RPN, RPN)],
                        src.at[pl.ds(sid * RPN, RPN)])

        @pl.loop(0, RPN // 125)
        def _(i):
            pltpu.sync_copy(zbuf, acc.at[pl.ds(sid * RPN + i * 125, 125)])

        plsc.subcore_barrier()

        gather(0, 0).start()

        @pl.loop(0, NCHUNK, step=2)
        def _(j):
            for b in range(2):
                jj = j + b

                @pl.when(jj >= 1)
                def _():
                    scatter(jj - 1, 1 - b).wait()

                @pl.when(jj + 1 < NCHUNK)
                def _():
                    gather(jj + 1, 1 - b).start()

                if p == 0:
                    coeffs(jj)

                gather(jj, b).wait()
                rv = rows[b]

                @pl.loop(0, K // 16)
                def _(v):
                    cv = c2d[jj, pl.ds(v * 16, 16)]
                    base = v * 16
                    for l in range(16):
                        cj = cv[l]
                        for d in range(DQ // 16):
                            sl = pl.ds(d * 16, 16)
                            rv[base + l, sl] = rv[base + l, sl] * cj

                scatter(jj, b).start(add=True)

        scatter(NCHUNK - 1, 1).wait()

        plsc.subcore_barrier()
        pltpu.sync_copy(acc.at[pl.ds(sid * RPN, RPN)],
                        out_hbm.at[qid, pl.ds(sid * RPN, RPN)])


BLK = 2000
GRID = N // BLK


def _tcdeg_body(degp_ref, deg_ref):
    deg_ref[...] = (jnp.sum(degp_ref[...], axis=(0, 1)) + 1.0)[:, None]


def _tcdeg(degp):
    return pl.pallas_call(
        _tcdeg_body,
        grid=(1,),
        in_specs=[pl.BlockSpec((NC, NS, NP), lambda i: (0, 0, 0))],
        out_specs=pl.BlockSpec((NP, 1), lambda i: (0, 0)),
        out_shape=jax.ShapeDtypeStruct((NP, 1), jnp.float32),
    )(degp)


def _tc1_body(x_ref, deg_ref, att2_ref,
              al_ref, ar_ref, dis_ref, s1_ref, xq_ref):
    x = x_ref[...]
    aa = jnp.dot(x, att2_ref[...], preferred_element_type=jnp.float32)
    al = aa[:, 0]
    ar = aa[:, 1]
    deg = deg_ref[:, 0]
    dis = lax.rsqrt(deg)
    s1 = jnp.tanh(al + ar) / deg
    al_ref[...] = al[:, None]
    ar_ref[...] = ar[:, None]
    dis_ref[...] = dis[:, None]
    s1_ref[...] = s1[:, None]
    xs = x * dis[:, None]
    for q in range(4):
        xq_ref[q] = xs[:, q * DQ:(q + 1) * DQ]


def _tc1(x, deg, att2):
    return pl.pallas_call(
        _tc1_body,
        grid=(GRID,),
        in_specs=[
            pl.BlockSpec((BLK, D), lambda i: (i, 0)),
            pl.BlockSpec((BLK, 1), lambda i: (i, 0)),
            pl.BlockSpec((D, 2), lambda i: (0, 0)),
        ],
        out_specs=[
            pl.BlockSpec((BLK, 1), lambda i: (i, 0)),
            pl.BlockSpec((BLK, 1), lambda i: (i, 0)),
            pl.BlockSpec((BLK, 1), lambda i: (i, 0)),
            pl.BlockSpec((BLK, 1), lambda i: (i, 0)),
            pl.BlockSpec((4, BLK, DQ), lambda i: (0, i, 0)),
        ],
        out_shape=[jax.ShapeDtypeStruct((N, 1), jnp.float32)] * 4
        + [jax.ShapeDtypeStruct((4, N, DQ), jnp.float32)],
    )(x, deg, att2)


def _tcd1_body(p_ref, x_ref, s1_ref, dis_ref, att2_ref,
               trgq_ref, al2_ref, ar2_ref, s2_ref):
    dis = dis_ref[...]
    c0 = (s1_ref[...] + EPS) * x_ref[...]
    trg = dis * jnp.concatenate([p_ref[q] for q in range(4)], axis=1) + c0
    aa = jnp.dot(trg, att2_ref[...], preferred_element_type=jnp.float32)
    al2 = aa[:, 0]
    ar2 = aa[:, 1]
    d1 = dis[:, 0]
    s2 = jnp.tanh(al2 + ar2) * d1
    al2_ref[...] = al2[:, None]
    ar2_ref[...] = ar2[:, None]
    s2_ref[...] = s2[:, None]
    trgs = trg * dis
    for q in range(4):
        trgq_ref[q] = trgs[:, q * DQ:(q + 1) * DQ]


def _tcd1(p, x, s1, dis, att2):
    return pl.pallas_call(
        _tcd1_body,
        grid=(GRID,),
        in_specs=[
            pl.BlockSpec((4, BLK, DQ), lambda i: (0, i, 0)),
            pl.BlockSpec((BLK, D), lambda i: (i, 0)),
            pl.BlockSpec((BLK, 1), lambda i: (i, 0)),
            pl.BlockSpec((BLK, 1), lambda i: (i, 0)),
            pl.BlockSpec((D, 2), lambda i: (0, 0)),
        ],
        out_specs=[
            pl.BlockSpec((4, BLK, DQ), lambda i: (0, i, 0)),
            pl.BlockSpec((BLK, 1), lambda i: (i, 0)),
            pl.BlockSpec((BLK, 1), lambda i: (i, 0)),
            pl.BlockSpec((BLK, 1), lambda i: (i, 0)),
        ],
        out_shape=[
            jax.ShapeDtypeStruct((4, N, DQ), jnp.float32),
            jax.ShapeDtypeStruct((N, 1), jnp.float32),
            jax.ShapeDtypeStruct((N, 1), jnp.float32),
            jax.ShapeDtypeStruct((N, 1), jnp.float32),
        ],
    )(p, x, s1, dis, att2)


def _tcd2_body(q_ref, trgq_ref, x_ref, s2_ref, dis_ref, wt_ref, b_ref,
               out_ref):
    trgs = jnp.concatenate([trgq_ref[i] for i in range(4)], axis=1)
    q = jnp.concatenate([q_ref[i] for i in range(4)], axis=1)
    h2 = dis_ref[...] * q + s2_ref[...] * trgs + EPS * x_ref[...]
    out_ref[...] = (jnp.dot(h2, wt_ref[...], preferred_element_type=jnp.float32)
                    + b_ref[...])


def _tcd2(q, trgq, x, s2, dis, wt, b):
    return pl.pallas_call(
        _tcd2_body,
        grid=(GRID,),
        in_specs=[
            pl.BlockSpec((4, BLK, DQ), lambda i: (0, i, 0)),
            pl.BlockSpec((4, BLK, DQ), lambda i: (0, i, 0)),
            pl.BlockSpec((BLK, D), lambda i: (i, 0)),
            pl.BlockSpec((BLK, 1), lambda i: (i, 0)),
            pl.BlockSpec((BLK, 1), lambda i: (i, 0)),
            pl.BlockSpec((D, 2), lambda i: (0, 0)),
            pl.BlockSpec((1, 2), lambda i: (0, 0)),
        ],
        out_specs=pl.BlockSpec((BLK, 2), lambda i: (i, 0)),
        out_shape=jax.ShapeDtypeStruct((N, 2), jnp.float32),
    )(q, trgq, x, s2, dis, wt, b)


def kernel(x, trg_edge, att_l, att_r, W_pred, b_pred):
    edge4d = trg_edge.reshape(2, NS, NCHUNK, K)
    att2 = jnp.stack([att_l, att_r], axis=1)
    degp = _sc_deg(edge4d)
    deg = _tcdeg(degp)
    al, ar, dis, s1, xq = _tc1(x, deg, att2)
    p = _sc_edge(xq, edge4d, al.reshape(N), ar.reshape(N))
    trgq, al2, ar2, s2 = _tcd1(p, x, s1, dis, att2)
    q = _sc_edge(trgq, edge4d, al2.reshape(N), ar2.reshape(N))
    return _tcd2(q, trgq, x, s2, dis, W_pred.T, b_pred.reshape(1, 2))


# speedup vs baseline: 38.6253x; 1.1733x over previous
import functools

import jax
import jax.numpy as jnp
from jax import lax
from jax.experimental import pallas as pl
from jax.experimental.pallas import tpu as pltpu
from jax.experimental.pallas import tpu_sc as plsc

N = 10000
E = 320000
D = 128
DQ = D // 4
EPS = 0.5

NC = 2
NS = 16
EPW = E // NS
K = 80
NCHUNK = EPW // K
NP = 10240
RPT = NP // NS
RPN = N // NS

_MESH = plsc.VectorSubcoreMesh(core_axis_name="c", subcore_axis_name="s")
_SC_PARAMS = pltpu.CompilerParams(needs_layout_passes=False,
                                  use_tc_tiling_on_sc=False)


def _stable_tanh(z):
    e = jnp.exp(-2.0 * jnp.abs(z))
    return jnp.sign(z) * (1.0 - e) / (1.0 + e)


@functools.partial(
    pl.kernel,
    out_type=jax.ShapeDtypeStruct((NC, NS, NP), jnp.float32),
    mesh=_MESH,
    compiler_params=_SC_PARAMS,
    scratch_types=[
        pltpu.VMEM((NCHUNK, K), jnp.int32),
        pltpu.VMEM((NP,), jnp.float32),
    ],
)
def _sc_deg(edge_hbm, out_hbm, col2d, tbl):
    cid = lax.axis_index("c")
    sid = lax.axis_index("s")

    pltpu.sync_copy(edge_hbm.at[1, sid], col2d)

    zero16 = jnp.zeros((16,), jnp.float32)

    @pl.loop(0, NP // 16)
    def _(i):
        tbl[pl.ds(i * 16, 16)] = zero16

    jbase = cid * (NCHUNK // 2)

    @pl.loop(0, NCHUNK // 2)
    def _(j):
        for v in range(K // 16):
            ci = col2d[jbase + j, pl.ds(v * 16, 16)]
            cnt, last = plsc.scan_count(ci)
            plsc.addupdate_scatter(tbl, [ci], cnt.astype(jnp.float32),
                                   mask=last)

    pltpu.sync_copy(tbl, out_hbm.at[cid, sid])


@functools.partial(
    pl.kernel,
    out_type=jax.ShapeDtypeStruct((4, N, DQ), jnp.float32),
    mesh=_MESH,
    compiler_params=_SC_PARAMS,
    scratch_types=[
        pltpu.VMEM((NCHUNK, K), jnp.int32),
        pltpu.VMEM((NCHUNK, K), jnp.int32),
        pltpu.VMEM((N,), jnp.float32),
        pltpu.VMEM((N,), jnp.float32),
        pltpu.VMEM((NCHUNK, K), jnp.float32),
        pltpu.VMEM((K, DQ), jnp.float32),
        pltpu.VMEM((K, DQ), jnp.float32),
        pltpu.VMEM((K, DQ), jnp.float32),
        pltpu.VMEM((K, DQ), jnp.float32),
        pltpu.VMEM((25, DQ), jnp.float32),
        pltpu.VMEM_SHARED((N, DQ), jnp.float32),
        pltpu.VMEM_SHARED((N, DQ), jnp.float32),
        pltpu.SemaphoreType.DMA,
        pltpu.SemaphoreType.DMA,
        pltpu.SemaphoreType.DMA,
        pltpu.SemaphoreType.DMA,
        pltpu.SemaphoreType.DMA,
        pltpu.SemaphoreType.DMA,
        pltpu.SemaphoreType.DMA,
        pltpu.SemaphoreType.DMA,
    ],
)
def _sc_edge(xq_hbm, edge_hbm, al_hbm, ar_hbm, out_hbm,
             row2d, col2d, al_v, ar_v, c2d, rows0, rows1, rows2, rows3,
             zbuf, acc, src,
             gsem0, gsem1, gsem2, gsem3, ssem0, ssem1, ssem2, ssem3):
    cid = lax.axis_index("c")
    sid = lax.axis_index("s")

    rows = (rows0, rows1, rows2, rows3)
    gsem = (gsem0, gsem1, gsem2, gsem3)
    ssem = (ssem0, ssem1, ssem2, ssem3)

    pltpu.sync_copy(edge_hbm.at[0, sid], row2d)
    pltpu.sync_copy(edge_hbm.at[1, sid], col2d)
    pltpu.sync_copy(al_hbm, al_v)
    pltpu.sync_copy(ar_hbm, ar_v)

    zero16 = jnp.zeros((16,), jnp.float32)

    @pl.loop(0, 25)
    def _(i):
        for d in range(DQ // 16):
            zbuf[i, pl.ds(d * 16, 16)] = zero16

    def coeffs(j):
        for v in range(K // 16):
            sl = pl.ds(v * 16, 16)
            ri = row2d[j, sl]
            ci = col2d[j, sl]
            av = plsc.load_gather(al_v, [ri])
            bv = plsc.load_gather(ar_v, [ci])
            c2d[j, sl] = _stable_tanh(av + bv)

    def gather(j, b):
        return pltpu.make_async_copy(src.at[row2d.at[j]], rows[b], gsem[b])

    def scatter(j, b):
        return pltpu.make_async_copy(rows[b], acc.at[col2d.at[j]], ssem[b])

    for p in range(2):
        qid = 2 * cid + p

        pltpu.sync_copy(xq_hbm.at[qid, pl.ds(sid * RPN, RPN)],
                        src.at[pl.ds(sid * RPN, RPN)])

        @pl.loop(0, RPN // 25)
        def _(i):
            pltpu.sync_copy(zbuf, acc.at[pl.ds(sid * RPN + i * 25, 25)])

        plsc.subcore_barrier()

        def process(jj, b):
            nb = (b + 1) % 4

            @pl.when(jj >= 3)
            def _():
                scatter(jj - 3, nb).wait()

            @pl.when(jj + 1 < NCHUNK)
            def _():
                gather(jj + 1, nb).start()

            if p == 0:
                coeffs(jj)

            gather(jj, b).wait()
            rv = rows[b]

            @pl.loop(0, K // 16)
            def _(v):
                cv = c2d[jj, pl.ds(v * 16, 16)]
                base = v * 16
                for l in range(16):
                    cj = cv[l]
                    for d in range(DQ // 16):
                        sl = pl.ds(d * 16, 16)
                        rv[base + l, sl] = rv[base + l, sl] * cj

            scatter(jj, b).start(add=True)

        gather(0, 0).start()

        @pl.loop(0, NCHUNK - 2, step=4)
        def _(j):
            for b in range(4):
                process(j + b, b)

        process(NCHUNK - 2, 0)
        process(NCHUNK - 1, 1)

        scatter(NCHUNK - 3, 3).wait()
        scatter(NCHUNK - 2, 0).wait()
        scatter(NCHUNK - 1, 1).wait()

        plsc.subcore_barrier()
        pltpu.sync_copy(acc.at[pl.ds(sid * RPN, RPN)],
                        out_hbm.at[qid, pl.ds(sid * RPN, RPN)])


BLK = 2000
GRID = N // BLK


def _tcdeg_body(degp_ref, deg_ref):
    deg_ref[...] = (jnp.sum(degp_ref[...], axis=(0, 1)) + 1.0)[:, None]


def _tcdeg(degp):
    return pl.pallas_call(
        _tcdeg_body,
        grid=(1,),
        in_specs=[pl.BlockSpec((NC, NS, NP), lambda i: (0, 0, 0))],
        out_specs=pl.BlockSpec((NP, 1), lambda i: (0, 0)),
        out_shape=jax.ShapeDtypeStruct((NP, 1), jnp.float32),
    )(degp)


def _tc1_body(x_ref, deg_ref, att2_ref,
              al_ref, ar_ref, dis_ref, s1_ref, xq_ref):
    x = x_ref[...]
    aa = jnp.dot(x, att2_ref[...], preferred_element_type=jnp.float32)
    al = aa[:, 0]
    ar = aa[:, 1]
    deg = deg_ref[:, 0]
    dis = lax.rsqrt(deg)
    s1 = jnp.tanh(al + ar) / deg
    al_ref[...] = al[:, None]
    ar_ref[...] = ar[:, None]
    dis_ref[...] = dis[:, None]
    s1_ref[...] = s1[:, None]
    xs = x * dis[:, None]
    for q in range(4):
        xq_ref[q] = xs[:, q * DQ:(q + 1) * DQ]


def _tc1(x, deg, att2):
    return pl.pallas_call(
        _tc1_body,
        grid=(GRID,),
        in_specs=[
            pl.BlockSpec((BLK, D), lambda i: (i, 0)),
            pl.BlockSpec((BLK, 1), lambda i: (i, 0)),
            pl.BlockSpec((D, 2), lambda i: (0, 0)),
        ],
        out_specs=[
            pl.BlockSpec((BLK, 1), lambda i: (i, 0)),
            pl.BlockSpec((BLK, 1), lambda i: (i, 0)),
            pl.BlockSpec((BLK, 1), lambda i: (i, 0)),
            pl.BlockSpec((BLK, 1), lambda i: (i, 0)),
            pl.BlockSpec((4, BLK, DQ), lambda i: (0, i, 0)),
        ],
        out_shape=[jax.ShapeDtypeStruct((N, 1), jnp.float32)] * 4
        + [jax.ShapeDtypeStruct((4, N, DQ), jnp.float32)],
    )(x, deg, att2)


def _tcd1_body(p_ref, x_ref, s1_ref, dis_ref, att2_ref,
               trgq_ref, al2_ref, ar2_ref, s2_ref):
    dis = dis_ref[...]
    c0 = (s1_ref[...] + EPS) * x_ref[...]
    trg = dis * jnp.concatenate([p_ref[q] for q in range(4)], axis=1) + c0
    aa = jnp.dot(trg, att2_ref[...], preferred_element_type=jnp.float32)
    al2 = aa[:, 0]
    ar2 = aa[:, 1]
    d1 = dis[:, 0]
    s2 = jnp.tanh(al2 + ar2) * d1
    al2_ref[...] = al2[:, None]
    ar2_ref[...] = ar2[:, None]
    s2_ref[...] = s2[:, None]
    trgs = trg * dis
    for q in range(4):
        trgq_ref[q] = trgs[:, q * DQ:(q + 1) * DQ]


def _tcd1(p, x, s1, dis, att2):
    return pl.pallas_call(
        _tcd1_body,
        grid=(GRID,),
        in_specs=[
            pl.BlockSpec((4, BLK, DQ), lambda i: (0, i, 0)),
            pl.BlockSpec((BLK, D), lambda i: (i, 0)),
            pl.BlockSpec((BLK, 1), lambda i: (i, 0)),
            pl.BlockSpec((BLK, 1), lambda i: (i, 0)),
            pl.BlockSpec((D, 2), lambda i: (0, 0)),
        ],
        out_specs=[
            pl.BlockSpec((4, BLK, DQ), lambda i: (0, i, 0)),
            pl.BlockSpec((BLK, 1), lambda i: (i, 0)),
            pl.BlockSpec((BLK, 1), lambda i: (i, 0)),
            pl.BlockSpec((BLK, 1), lambda i: (i, 0)),
        ],
        out_shape=[
            jax.ShapeDtypeStruct((4, N, DQ), jnp.float32),
            jax.ShapeDtypeStruct((N, 1), jnp.float32),
            jax.ShapeDtypeStruct((N, 1), jnp.float32),
            jax.ShapeDtypeStruct((N, 1), jnp.float32),
        ],
    )(p, x, s1, dis, att2)


def _tcd2_body(q_ref, trgq_ref, x_ref, s2_ref, dis_ref, wt_ref, b_ref,
               out_ref):
    trgs = jnp.concatenate([trgq_ref[i] for i in range(4)], axis=1)
    q = jnp.concatenate([q_ref[i] for i in range(4)], axis=1)
    h2 = dis_ref[...] * q + s2_ref[...] * trgs + EPS * x_ref[...]
    out_ref[...] = (jnp.dot(h2, wt_ref[...], preferred_element_type=jnp.float32)
                    + b_ref[...])


def _tcd2(q, trgq, x, s2, dis, wt, b):
    return pl.pallas_call(
        _tcd2_body,
        grid=(GRID,),
        in_specs=[
            pl.BlockSpec((4, BLK, DQ), lambda i: (0, i, 0)),
            pl.BlockSpec((4, BLK, DQ), lambda i: (0, i, 0)),
            pl.BlockSpec((BLK, D), lambda i: (i, 0)),
            pl.BlockSpec((BLK, 1), lambda i: (i, 0)),
            pl.BlockSpec((BLK, 1), lambda i: (i, 0)),
            pl.BlockSpec((D, 2), lambda i: (0, 0)),
            pl.BlockSpec((1, 2), lambda i: (0, 0)),
        ],
        out_specs=pl.BlockSpec((BLK, 2), lambda i: (i, 0)),
        out_shape=jax.ShapeDtypeStruct((N, 2), jnp.float32),
    )(q, trgq, x, s2, dis, wt, b)


def kernel(x, trg_edge, att_l, att_r, W_pred, b_pred):
    edge4d = trg_edge.reshape(2, NS, NCHUNK, K)
    att2 = jnp.stack([att_l, att_r], axis=1)
    degp = _sc_deg(edge4d)
    deg = _tcdeg(degp)
    al, ar, dis, s1, xq = _tc1(x, deg, att2)
    p = _sc_edge(xq, edge4d, al.reshape(N), ar.reshape(N))
    trgq, al2, ar2, s2 = _tcd1(p, x, s1, dis, att2)
    q = _sc_edge(trgq, edge4d, al2.reshape(N), ar2.reshape(N))
    return _tcd2(q, trgq, x, s2, dis, W_pred.T, b_pred.reshape(1, 2))
